# baseline (speedup 1.0000x reference)
import functools

import jax
import jax.numpy as jnp
from jax import lax
from jax.experimental import pallas as pl
from jax.experimental.pallas import tpu as pltpu

T, D, I, E, K = 2048, 1024, 512, 8, 2
BT = 256


def _dot_t(a, b):
    return lax.dot_general(a, b, (((1,), (1,)), ((), ())))


def _dot_t_bf16(a, b):
    return lax.dot_general(a.astype(jnp.bfloat16), b.astype(jnp.bfloat16),
                           (((1,), (1,)), ((), ())),
                           preferred_element_type=jnp.float32)


def _moe_body(x_ref, gate_w_ref, gp_ref, up_ref, dp_ref, sg_ref, su_ref,
              sd_ref, out_ref, logits_ref, ids_ref, acc_ref):
    g = pl.program_id(0)
    tb = pl.program_id(1)
    xb = x_ref[...]
    iota_e = lax.broadcasted_iota(jnp.int32, (BT, E), 1)

    logits = _dot_t(xb, gate_w_ref[...])
    m = jnp.max(logits, axis=1, keepdims=True)
    p = jnp.exp(logits - m)
    probs = p / jnp.sum(p, axis=1, keepdims=True)
    m1 = jnp.max(probs, axis=1, keepdims=True)
    a1 = jnp.min(jnp.where(probs == m1, iota_e, E), axis=1, keepdims=True)
    probs2 = jnp.where(iota_e == a1, -1.0, probs)
    m2 = jnp.max(probs2, axis=1, keepdims=True)
    a2 = jnp.min(jnp.where(probs2 == m2, iota_e, E), axis=1, keepdims=True)
    s = m1 + m2 + 1e-9
    comb = (jnp.where(iota_e == a1, m1 / s, 0.0)
            + jnp.where(iota_e == a2, m2 / s, 0.0))
    row = tb * BT

    @pl.when(g == 0)
    def _router_out():
        logits_ref[...] = logits
        ids_ref[...] = jnp.concatenate([a1, a2], axis=1)

    @pl.when(g == 0)
    def _shared():
        hg = _dot_t_bf16(xb, sg_ref[...])
        hu = _dot_t_bf16(xb, su_ref[...])
        h = hg / (1.0 + jnp.exp(-hg)) * hu
        acc_ref[pl.ds(row, BT), :] = _dot_t_bf16(h, sd_ref[...])

    @pl.when(g > 0)
    def _expert():
        hg = _dot_t_bf16(xb, gp_ref[0])
        hu = _dot_t_bf16(xb, up_ref[0])
        h = hg / (1.0 + jnp.exp(-hg)) * hu
        eo = _dot_t_bf16(h, dp_ref[0])
        w = jnp.sum(comb * (iota_e == (g - 1)).astype(jnp.float32),
                    axis=1, keepdims=True)
        acc_ref[pl.ds(row, BT), :] += w * eo

    @pl.when(g == E)
    def _emit():
        out_ref[...] = acc_ref[pl.ds(row, BT), :]


@jax.jit
def _moe_dense(x, gate_w, gp, up, dp, sg, su, sd):
    grid = (E + 1, T // BT)

    def _wmap(g, tb):
        return (jnp.maximum(g - 1, 0), 0, 0)

    return pl.pallas_call(
        _moe_body,
        grid=grid,
        in_specs=[
            pl.BlockSpec((BT, D), lambda g, tb: (tb, 0)),
            pl.BlockSpec((E, D), lambda g, tb: (0, 0)),
            pl.BlockSpec((1, I, D), _wmap),
            pl.BlockSpec((1, I, D), _wmap),
            pl.BlockSpec((1, D, I), _wmap),
            pl.BlockSpec((I, D), lambda g, tb: (0, 0)),
            pl.BlockSpec((I, D), lambda g, tb: (0, 0)),
            pl.BlockSpec((D, I), lambda g, tb: (0, 0)),
        ],
        out_specs=[
            pl.BlockSpec((BT, D), lambda g, tb: (jnp.where(g == E, tb, 0), 0)),
            pl.BlockSpec((BT, E),
                         lambda g, tb: (jnp.where(g == 0, tb, T // BT - 1), 0)),
            pl.BlockSpec((BT, K),
                         lambda g, tb: (jnp.where(g == 0, tb, T // BT - 1), 0)),
        ],
        out_shape=[
            jax.ShapeDtypeStruct((T, D), jnp.float32),
            jax.ShapeDtypeStruct((T, E), jnp.float32),
            jax.ShapeDtypeStruct((T, K), jnp.int32),
        ],
        scratch_shapes=[
            pltpu.VMEM((T, D), jnp.float32),
        ],
    )(x, gate_w, gp, up, dp, sg, su, sd)


def kernel(hidden_state, gate_w, gate_proj, up_proj, down_proj, shared_gate,
           shared_up, shared_down):
    Bv, Nv, Dv = hidden_state.shape
    x = hidden_state.reshape(Bv * Nv, Dv)
    out, logits, ids = _moe_dense(x, gate_w, gate_proj, up_proj, down_proj,
                                  shared_gate, shared_up, shared_down)
    return out.reshape(Bv, Nv, Dv), logits, ids


# speedup vs baseline: 1.2131x; 1.2131x over previous
import functools

import jax
import jax.numpy as jnp
from jax import lax
from jax.experimental import pallas as pl
from jax.experimental.pallas import tpu as pltpu
from jax.experimental.pallas import tpu_sc as plsc

T, D, I, E, K = 2048, 1024, 512, 8, 2
BM = 256
S = 6144
NBLK = S // BM
NC, NS = 2, 16
NW = NC * NS
CH = 32
NCHUNK = T // (NW * CH)


def _dot_t(a, b):
    return lax.dot_general(a, b, (((1,), (1,)), ((), ())))


def _dot_t_bf16(a, b):
    return lax.dot_general(a.astype(jnp.bfloat16), b.astype(jnp.bfloat16),
                           (((1,), (1,)), ((), ())),
                           preferred_element_type=jnp.float32)


def _silu(x):
    return x / (1.0 + jnp.exp(-x))


def _router_body(x_ref, gate_w_ref, sg_ref, su_ref, sd_ref,
                 sh_ref, logits_ref, ids_ref, pos0_ref, pos1_ref,
                 w0_ref, w1_ref, be_ref):
    x = x_ref[...]
    logits = _dot_t(x, gate_w_ref[...])
    logits_ref[...] = logits
    m = jnp.max(logits, axis=1, keepdims=True)
    p = jnp.exp(logits - m)
    probs = p / jnp.sum(p, axis=1, keepdims=True)
    iota_e = lax.broadcasted_iota(jnp.int32, (T, E), 1)
    m1 = jnp.max(probs, axis=1, keepdims=True)
    a1 = jnp.min(jnp.where(probs == m1, iota_e, E), axis=1, keepdims=True)
    probs2 = jnp.where(iota_e == a1, -1.0, probs)
    m2 = jnp.max(probs2, axis=1, keepdims=True)
    a2 = jnp.min(jnp.where(probs2 == m2, iota_e, E), axis=1, keepdims=True)
    s = m1 + m2 + 1e-9
    ids_ref[...] = jnp.concatenate([a1, a2], axis=1)
    w0_ref[...] = jnp.broadcast_to(m1 / s, (T, 128))
    w1_ref[...] = jnp.broadcast_to(m2 / s, (T, 128))

    h1 = (iota_e == a1).astype(jnp.int32)
    h2 = (iota_e == a2).astype(jnp.int32)
    h = h1 + h2
    c = h
    sh = 1
    while sh < T:
        c = c + jnp.concatenate(
            [jnp.zeros((sh, E), jnp.int32), c[:T - sh]], axis=0)
        sh *= 2
    cexcl = c - h
    counts = c[T - 1:T, :]
    pc = ((counts + (BM - 1)) // BM) * BM
    cp = pc
    sh = 1
    while sh < E:
        cp = cp + jnp.concatenate(
            [jnp.zeros((1, sh), jnp.int32), cp[:, :E - sh]], axis=1)
        sh *= 2
    offs = cp - pc
    dest = offs + cexcl
    pos0_ref[...] = jnp.sum(jnp.where(iota_e == a1, dest, 0), axis=1,
                            keepdims=True)
    pos1_ref[...] = jnp.sum(jnp.where(iota_e == a2, dest, 0), axis=1,
                            keepdims=True)
    iota_b = lax.broadcasted_iota(jnp.int32, (NBLK, E), 0) * BM
    be_ref[...] = jnp.sum((iota_b >= jnp.broadcast_to(cp, (NBLK, E)))
                          .astype(jnp.int32), axis=1, keepdims=True)

    hg = _dot_t_bf16(x, sg_ref[...])
    hu = _dot_t_bf16(x, su_ref[...])
    sh_ref[...] = _dot_t_bf16(_silu(hg) * hu, sd_ref[...])


def _router(x, gate_w, sg, su, sd):
    return pl.pallas_call(
        _router_body,
        out_shape=[
            jax.ShapeDtypeStruct((T, D), jnp.float32),
            jax.ShapeDtypeStruct((T, E), jnp.float32),
            jax.ShapeDtypeStruct((T, K), jnp.int32),
            jax.ShapeDtypeStruct((T, 1), jnp.int32),
            jax.ShapeDtypeStruct((T, 1), jnp.int32),
            jax.ShapeDtypeStruct((T, 128), jnp.float32),
            jax.ShapeDtypeStruct((T, 128), jnp.float32),
            jax.ShapeDtypeStruct((NBLK, 1), jnp.int32),
        ],
    )(x, gate_w, sg, su, sd)


def _dispatch(x, pos0r, pos1r, w0m, w1m):
    mesh = plsc.VectorSubcoreMesh(core_axis_name="c", subcore_axis_name="s")

    @functools.partial(
        pl.kernel, mesh=mesh,
        out_type=[jax.ShapeDtypeStruct((S, D), jnp.float32),
                  jax.ShapeDtypeStruct((S, 128), jnp.float32)],
        scratch_types=[
            pltpu.VMEM((CH,), jnp.int32),
            pltpu.VMEM((CH,), jnp.int32),
            pltpu.VMEM((CH, D), jnp.float32),
            pltpu.VMEM((CH, 128), jnp.float32),
            pltpu.VMEM((CH, 128), jnp.float32),
            pltpu.SemaphoreType.DMA,
        ],
    )
    def k(x_hbm, pos0_hbm, pos1_hbm, w0_hbm, w1_hbm, xs_hbm, ws_hbm,
          i0_v, i1_v, xr_v, w0_v, w1_v, sem):
        wid = lax.axis_index("s") * NC + lax.axis_index("c")
        for c in range(NCHUNK):
            row = wid * NCHUNK + c
            base = row * CH
            pltpu.sync_copy(pos0_hbm.at[row], i0_v)
            pltpu.sync_copy(pos1_hbm.at[row], i1_v)
            pltpu.sync_copy(x_hbm.at[pl.ds(base, CH)], xr_v)
            pltpu.sync_copy(w0_hbm.at[pl.ds(base, CH)], w0_v)
            pltpu.sync_copy(w1_hbm.at[pl.ds(base, CH)], w1_v)
            c1 = pltpu.async_copy(xr_v, xs_hbm.at[i0_v], sem)
            c2 = pltpu.async_copy(xr_v, xs_hbm.at[i1_v], sem)
            c3 = pltpu.async_copy(w0_v, ws_hbm.at[i0_v], sem)
            c4 = pltpu.async_copy(w1_v, ws_hbm.at[i1_v], sem)
            c1.wait(); c2.wait(); c3.wait(); c4.wait()

    return k(x, pos0r, pos1r, w0m, w1m)


def _gmlp_body(be_ref, xs_ref, ws_ref, gp_ref, up_ref, dp_ref, po_ref):
    b = pl.program_id(0)

    @pl.when(be_ref[b] < E)
    def _():
        xb = xs_ref[...]
        hg = _dot_t_bf16(xb, gp_ref[0])
        hu = _dot_t_bf16(xb, up_ref[0])
        h = _silu(hg) * hu
        po_ref[...] = ws_ref[:, 0:1] * _dot_t_bf16(h, dp_ref[0])


def _gmlp(blk_exp, xs, ws, gp, up, dp):
    def _wmap(b, be):
        return (jnp.minimum(be[b], E - 1), 0, 0)

    grid_spec = pltpu.PrefetchScalarGridSpec(
        num_scalar_prefetch=1,
        grid=(NBLK,),
        in_specs=[
            pl.BlockSpec((BM, D), lambda b, be: (b, 0)),
            pl.BlockSpec((BM, 128), lambda b, be: (b, 0)),
            pl.BlockSpec((1, I, D), _wmap),
            pl.BlockSpec((1, I, D), _wmap),
            pl.BlockSpec((1, D, I), _wmap),
        ],
        out_specs=pl.BlockSpec((BM, D), lambda b, be: (b, 0)),
    )
    return pl.pallas_call(
        _gmlp_body, grid_spec=grid_spec,
        out_shape=jax.ShapeDtypeStruct((S, D), jnp.float32),
    )(blk_exp, xs, ws, gp, up, dp)


def _combine(po, pos0r, pos1r, sh):
    mesh = plsc.VectorSubcoreMesh(core_axis_name="c", subcore_axis_name="s")

    @functools.partial(
        pl.kernel, mesh=mesh,
        out_type=jax.ShapeDtypeStruct((T, D), jnp.float32),
        scratch_types=[
            pltpu.VMEM((CH,), jnp.int32),
            pltpu.VMEM((CH,), jnp.int32),
            pltpu.VMEM((CH, D), jnp.float32),
            pltpu.VMEM((CH, D), jnp.float32),
            pltpu.VMEM((CH, D), jnp.float32),
            pltpu.SemaphoreType.DMA,
        ],
    )
    def k(po_hbm, pos0_hbm, pos1_hbm, sh_hbm, out_hbm,
          i0_v, i1_v, acc_v, g0_v, g1_v, sem):
        wid = lax.axis_index("s") * NC + lax.axis_index("c")
        nv = D // 16
        for c in range(NCHUNK):
            row = wid * NCHUNK + c
            base = row * CH
            pltpu.sync_copy(pos0_hbm.at[row], i0_v)
            pltpu.sync_copy(pos1_hbm.at[row], i1_v)
            pltpu.sync_copy(sh_hbm.at[pl.ds(base, CH)], acc_v)
            h0 = pltpu.async_copy(po_hbm.at[i0_v], g0_v, sem)
            h1 = pltpu.async_copy(po_hbm.at[i1_v], g1_v, sem)
            h0.wait()
            h1.wait()

            def addb(i, carry):
                r = i // nv
                cc = (i % nv) * 16
                acc_v[r, pl.ds(cc, 16)] = (acc_v[r, pl.ds(cc, 16)]
                                           + g0_v[r, pl.ds(cc, 16)]
                                           + g1_v[r, pl.ds(cc, 16)])
                return carry

            lax.fori_loop(0, CH * nv, addb, 0)
            pltpu.sync_copy(acc_v, out_hbm.at[pl.ds(base, CH)])

    return k(po, pos0r, pos1r, sh)


def kernel(hidden_state, gate_w, gate_proj, up_proj, down_proj, shared_gate,
           shared_up, shared_down):
    Bv, Nv, Dv = hidden_state.shape
    x = hidden_state.reshape(Bv * Nv, Dv)
    sh, logits, ids, pos0, pos1, w0m, w1m, be = _router(
        x, gate_w, shared_gate, shared_up, shared_down)
    pos0r = pos0.reshape(T // CH, CH)
    pos1r = pos1.reshape(T // CH, CH)
    xs, ws = _dispatch(x, pos0r, pos1r, w0m, w1m)
    po = _gmlp(be.reshape(NBLK), xs, ws, gate_proj, up_proj, down_proj)
    out = _combine(po, pos0r, pos1r, sh)
    return out.reshape(Bv, Nv, Dv), logits, ids


# speedup vs baseline: 1.3278x; 1.0946x over previous
import functools

import jax
import jax.numpy as jnp
from jax import lax
from jax.experimental import pallas as pl
from jax.experimental.pallas import tpu as pltpu
from jax.experimental.pallas import tpu_sc as plsc

T, D, I, E, K = 2048, 1024, 512, 8, 2
BM = 256
S = 6144
NBLK = S // BM
NC, NS = 2, 16
NW = NC * NS
CH = 32
NCHUNK = T // (NW * CH)


def _dot_t(a, b):
    return lax.dot_general(a, b, (((1,), (1,)), ((), ())))


def _dot_t_bf16(a, b):
    return lax.dot_general(a.astype(jnp.bfloat16), b.astype(jnp.bfloat16),
                           (((1,), (1,)), ((), ())),
                           preferred_element_type=jnp.float32)


def _silu(x):
    return x / (1.0 + jnp.exp(-x))


def _router_body(x_ref, gate_w_ref, sg_ref, su_ref, sd_ref,
                 sh_ref, logits_ref, ids_ref, pos0_ref, pos1_ref,
                 w0_ref, w1_ref, be_ref):
    x = x_ref[...]
    logits = _dot_t(x, gate_w_ref[...])
    logits_ref[...] = logits
    m = jnp.max(logits, axis=1, keepdims=True)
    p = jnp.exp(logits - m)
    probs = p / jnp.sum(p, axis=1, keepdims=True)
    iota_e = lax.broadcasted_iota(jnp.int32, (T, E), 1)
    m1 = jnp.max(probs, axis=1, keepdims=True)
    a1 = jnp.min(jnp.where(probs == m1, iota_e, E), axis=1, keepdims=True)
    probs2 = jnp.where(iota_e == a1, -1.0, probs)
    m2 = jnp.max(probs2, axis=1, keepdims=True)
    a2 = jnp.min(jnp.where(probs2 == m2, iota_e, E), axis=1, keepdims=True)
    s = m1 + m2 + 1e-9
    ids_ref[...] = jnp.concatenate([a1, a2], axis=1)
    w0_ref[...] = jnp.broadcast_to(m1 / s, (T, 128))
    w1_ref[...] = jnp.broadcast_to(m2 / s, (T, 128))

    h1 = (iota_e == a1).astype(jnp.int32)
    h2 = (iota_e == a2).astype(jnp.int32)
    h = h1 + h2
    c = h
    sh = 1
    while sh < T:
        c = c + jnp.concatenate(
            [jnp.zeros((sh, E), jnp.int32), c[:T - sh]], axis=0)
        sh *= 2
    cexcl = c - h
    counts = c[T - 1:T, :]
    pc = ((counts + (BM - 1)) // BM) * BM
    cp = pc
    sh = 1
    while sh < E:
        cp = cp + jnp.concatenate(
            [jnp.zeros((1, sh), jnp.int32), cp[:, :E - sh]], axis=1)
        sh *= 2
    offs = cp - pc
    dest = offs + cexcl
    pos0_ref[...] = jnp.sum(jnp.where(iota_e == a1, dest, 0), axis=1,
                            keepdims=True)
    pos1_ref[...] = jnp.sum(jnp.where(iota_e == a2, dest, 0), axis=1,
                            keepdims=True)
    iota_b = lax.broadcasted_iota(jnp.int32, (NBLK, E), 0) * BM
    be_ref[...] = jnp.sum((iota_b >= jnp.broadcast_to(cp, (NBLK, E)))
                          .astype(jnp.int32), axis=1, keepdims=True)

    hg = _dot_t_bf16(x, sg_ref[...])
    hu = _dot_t_bf16(x, su_ref[...])
    sh_ref[...] = _dot_t_bf16(_silu(hg) * hu, sd_ref[...])


def _router(x, gate_w, sg, su, sd):
    return pl.pallas_call(
        _router_body,
        out_shape=[
            jax.ShapeDtypeStruct((T, D), jnp.float32),
            jax.ShapeDtypeStruct((T, E), jnp.float32),
            jax.ShapeDtypeStruct((T, K), jnp.int32),
            jax.ShapeDtypeStruct((T, 1), jnp.int32),
            jax.ShapeDtypeStruct((T, 1), jnp.int32),
            jax.ShapeDtypeStruct((T, 128), jnp.float32),
            jax.ShapeDtypeStruct((T, 128), jnp.float32),
            jax.ShapeDtypeStruct((NBLK, 1), jnp.int32),
        ],
    )(x, gate_w, sg, su, sd)


def _dispatch(x, pos0r, pos1r, w0m, w1m):
    mesh = plsc.VectorSubcoreMesh(core_axis_name="c", subcore_axis_name="s")

    @functools.partial(
        pl.kernel, mesh=mesh,
        out_type=[jax.ShapeDtypeStruct((S, D), jnp.float32),
                  jax.ShapeDtypeStruct((S, 128), jnp.float32)],
        scratch_types=[
            pltpu.VMEM((NCHUNK, CH), jnp.int32),
            pltpu.VMEM((NCHUNK, CH), jnp.int32),
            pltpu.VMEM((NCHUNK, CH, D), jnp.float32),
            pltpu.VMEM((NCHUNK, CH, 128), jnp.float32),
            pltpu.VMEM((NCHUNK, CH, 128), jnp.float32),
            pltpu.SemaphoreType.DMA,
        ],
    )
    def k(x_hbm, pos0_hbm, pos1_hbm, w0_hbm, w1_hbm, xs_hbm, ws_hbm,
          i0_v, i1_v, xr_v, w0_v, w1_v, sem):
        wid = lax.axis_index("s") * NC + lax.axis_index("c")
        handles = []
        for c in range(NCHUNK):
            row = wid * NCHUNK + c
            base = row * CH
            pltpu.sync_copy(pos0_hbm.at[row], i0_v.at[c])
            pltpu.sync_copy(pos1_hbm.at[row], i1_v.at[c])
            pltpu.sync_copy(x_hbm.at[pl.ds(base, CH)], xr_v.at[c])
            pltpu.sync_copy(w0_hbm.at[pl.ds(base, CH)], w0_v.at[c])
            pltpu.sync_copy(w1_hbm.at[pl.ds(base, CH)], w1_v.at[c])
            handles.append(
                pltpu.async_copy(xr_v.at[c], xs_hbm.at[i0_v.at[c]], sem))
            handles.append(
                pltpu.async_copy(xr_v.at[c], xs_hbm.at[i1_v.at[c]], sem))
            handles.append(
                pltpu.async_copy(w0_v.at[c], ws_hbm.at[i0_v.at[c]], sem))
            handles.append(
                pltpu.async_copy(w1_v.at[c], ws_hbm.at[i1_v.at[c]], sem))
        for h in handles:
            h.wait()

    return k(x, pos0r, pos1r, w0m, w1m)


def _gmlp_body(be_ref, xs_ref, ws_ref, gp_ref, up_ref, dp_ref, po_ref):
    b = pl.program_id(0)

    @pl.when(be_ref[b] < E)
    def _():
        xb = xs_ref[...]
        hg = _dot_t_bf16(xb, gp_ref[0])
        hu = _dot_t_bf16(xb, up_ref[0])
        h = _silu(hg) * hu
        po_ref[...] = ws_ref[:, 0:1] * _dot_t_bf16(h, dp_ref[0])


def _gmlp(blk_exp, xs, ws, gp, up, dp):
    def _wmap(b, be):
        return (jnp.minimum(be[b], E - 1), 0, 0)

    grid_spec = pltpu.PrefetchScalarGridSpec(
        num_scalar_prefetch=1,
        grid=(NBLK,),
        in_specs=[
            pl.BlockSpec((BM, D), lambda b, be: (b, 0)),
            pl.BlockSpec((BM, 128), lambda b, be: (b, 0)),
            pl.BlockSpec((1, I, D), _wmap),
            pl.BlockSpec((1, I, D), _wmap),
            pl.BlockSpec((1, D, I), _wmap),
        ],
        out_specs=pl.BlockSpec((BM, D), lambda b, be: (b, 0)),
    )
    return pl.pallas_call(
        _gmlp_body, grid_spec=grid_spec,
        out_shape=jax.ShapeDtypeStruct((S, D), jnp.float32),
    )(blk_exp, xs, ws, gp, up, dp)


CCH = 16
CNCH = T // (NW * CCH)


def _combine(po, pos0r, pos1r, sh):
    mesh = plsc.VectorSubcoreMesh(core_axis_name="c", subcore_axis_name="s")

    @functools.partial(
        pl.kernel, mesh=mesh,
        out_type=jax.ShapeDtypeStruct((T, D), jnp.float32),
        scratch_types=[
            pltpu.VMEM((2, CCH), jnp.int32),
            pltpu.VMEM((2, CCH), jnp.int32),
            pltpu.VMEM((CCH, D), jnp.float32),
            pltpu.VMEM((2, CCH, D), jnp.float32),
            pltpu.VMEM((2, CCH, D), jnp.float32),
            pltpu.SemaphoreType.DMA,
            pltpu.SemaphoreType.DMA,
        ],
    )
    def k(po_hbm, pos0_hbm, pos1_hbm, sh_hbm, out_hbm,
          i0_v, i1_v, acc_v, g0_v, g1_v, sem_a, sem_b):
        wid = lax.axis_index("s") * NC + lax.axis_index("c")
        nv = D // 16
        sems = (sem_a, sem_b)

        def fire(c):
            pr = c % 2
            row = wid * CNCH + c
            pltpu.sync_copy(pos0_hbm.at[row], i0_v.at[pr])
            pltpu.sync_copy(pos1_hbm.at[row], i1_v.at[pr])
            return (pltpu.async_copy(po_hbm.at[i0_v.at[pr]], g0_v.at[pr],
                                     sems[pr]),
                    pltpu.async_copy(po_hbm.at[i1_v.at[pr]], g1_v.at[pr],
                                     sems[pr]))

        pending = fire(0)
        for c in range(CNCH):
            pr = c % 2
            base = (wid * CNCH + c) * CCH
            pltpu.sync_copy(sh_hbm.at[pl.ds(base, CCH)], acc_v)
            h0, h1 = pending
            h0.wait()
            h1.wait()
            if c + 1 < CNCH:
                pending = fire(c + 1)

            def addrow(r, carry):
                for v in range(nv):
                    sl = pl.ds(v * 16, 16)
                    plsc.addupdate(acc_v.at[r, sl],
                                   g0_v[pr, r, sl] + g1_v[pr, r, sl])
                return carry

            lax.fori_loop(0, CCH, addrow, 0)
            pltpu.sync_copy(acc_v, out_hbm.at[pl.ds(base, CCH)])

    return k(po, pos0r, pos1r, sh)


def kernel(hidden_state, gate_w, gate_proj, up_proj, down_proj, shared_gate,
           shared_up, shared_down):
    Bv, Nv, Dv = hidden_state.shape
    x = hidden_state.reshape(Bv * Nv, Dv)
    sh, logits, ids, pos0, pos1, w0m, w1m, be = _router(
        x, gate_w, shared_gate, shared_up, shared_down)
    xs, ws = _dispatch(x, pos0.reshape(T // CH, CH),
                       pos1.reshape(T // CH, CH), w0m, w1m)
    po = _gmlp(be.reshape(NBLK), xs, ws, gate_proj, up_proj, down_proj)
    out = _combine(po, pos0.reshape(T // CCH, CCH),
                   pos1.reshape(T // CCH, CCH), sh)
    return out.reshape(Bv, Nv, Dv), logits, ids
